# baseline (speedup 1.0000x reference)
import functools

import jax
import jax.numpy as jnp
from jax import lax
from jax.experimental import pallas as pl
from jax.experimental.pallas import tpu as pltpu
from jax.experimental.pallas import tpu_sc as plsc

_NUM_BINS = 1000
_D = 64
_BATCH = 16384
_HIST = 200
_TOTAL = _BATCH * _HIST

_NC = 2
_NS = 16
_NW = _NC * _NS
_CW = _BATCH // _NW
_G = 128
_NJ = _CW // _G
_NB = _HIST
_HT = _HIST // 8


def _body(iv_hbm, tab_hbm, out_hbm, iv, idx, rows,
          ivsem0, ivsem1, gsem0, gsem1, osem0, osem1):
    wid = lax.axis_index("s") * _NC + lax.axis_index("c")
    col0 = wid * _CW
    c0 = wid * _NJ
    ivsems = (ivsem0, ivsem1)
    gsems = (gsem0, gsem1)
    osems = (osem0, osem1)

    def iv_copies(h, b):
        t = h // 8
        s = h % 8
        return [
            (iv_hbm.at[t].at[c0 + j].at[s],
             iv.at[b].at[pl.ds(j * _G, _G)], ivsems[b])
            for j in range(_NJ)
        ]

    def fire_iv(h, b):
        for src, dst, sem in iv_copies(h, b):
            pltpu.async_copy(src, dst, sem)

    def stage(h, b):
        for src, dst, sem in iv_copies(h, b):
            pltpu.make_async_copy(src, dst, sem).wait()
        iv_b = iv.at[b]
        for j in range(_NJ):
            idx_bj = idx.at[b].at[j]
            for i in range(_G // 16):
                v = iv_b[pl.ds(j * _G + i * 16, 16)]
                bins = jnp.minimum(
                    lax.shift_right_logical(v * 37450, 18), _NUM_BINS - 1
                )
                idx_bj[pl.ds(i * 16, 16)] = bins
        for j in range(_NJ):
            pltpu.async_copy(
                tab_hbm.at[idx.at[b].at[j]],
                rows.at[b].at[pl.ds(j * _G, _G)],
                gsems[b],
            )

    def drain_gathers(b):
        for j in range(_NJ):
            pltpu.make_async_copy(
                tab_hbm.at[idx.at[b].at[j]],
                rows.at[b].at[pl.ds(j * _G, _G)],
                gsems[b],
            ).wait()

    def fire_out(h, b):
        pltpu.async_copy(
            rows.at[b],
            out_hbm.at[h].at[pl.ds(col0, _CW)],
            osems[b],
        )

    def drain_out(h, b):
        pltpu.make_async_copy(
            rows.at[b],
            out_hbm.at[h].at[pl.ds(col0, _CW)],
            osems[b],
        ).wait()

    fire_iv(0, 0)
    fire_iv(1, 1)
    stage(0, 0)
    stage(1, 1)

    def loop_body(i, carry):
        h = i * 2
        drain_gathers(0)
        fire_out(h, 0)
        drain_gathers(1)
        fire_out(h + 1, 1)

        @pl.when(h + 2 < _NB)
        def _():
            fire_iv(h + 2, 0)
            fire_iv(h + 3, 1)
            drain_out(h, 0)
            stage(h + 2, 0)
            drain_out(h + 1, 1)
            stage(h + 3, 1)

        return carry

    lax.fori_loop(0, _NB // 2, loop_body, 0)
    drain_out(_NB - 2, 0)
    drain_out(_NB - 1, 1)


_sc_lookup = functools.partial(
    pl.kernel,
    out_type=jax.ShapeDtypeStruct((_HIST, _BATCH, _D), jnp.float32),
    mesh=plsc.VectorSubcoreMesh(core_axis_name="c", subcore_axis_name="s"),
    compiler_params=pltpu.CompilerParams(use_tc_tiling_on_sc=False),
    scratch_types=[
        pltpu.VMEM((2, _CW), jnp.int32),
        pltpu.VMEM((2, _NJ, _G), jnp.int32),
        pltpu.VMEM((2, _CW, _D), jnp.float32),
        pltpu.SemaphoreType.DMA,
        pltpu.SemaphoreType.DMA,
        pltpu.SemaphoreType.DMA,
        pltpu.SemaphoreType.DMA,
        pltpu.SemaphoreType.DMA,
        pltpu.SemaphoreType.DMA,
    ],
)(_body)


def _impl(intervals, embed_weight):
    iv4 = intervals.reshape(128, 128, _HT, 8).transpose(2, 0, 3, 1)
    t_out = _sc_lookup(iv4, embed_weight)
    return t_out.transpose(1, 0, 2)


kernel = jax.jit(_impl)
